# baseline (speedup 1.0000x reference)
import functools

import jax
import jax.numpy as jnp
from jax import lax
from jax.experimental import pallas as pl
from jax.experimental.pallas import tpu as pltpu
from jax.experimental.pallas import tpu_sc as plsc

N_NODES = 10000
N_EDGES = 320000
D_FEAT = 128
D_EDGE = 16
D_GLOBAL = 16
D_OUT = D_EDGE + 2 * D_FEAT + D_GLOBAL
D_PAD = 384

_NC = 2
_NS = 16
_NW = _NC * _NS
_E_PER_W = N_EDGES // _NW
_B = 80
_STEPS = _E_PER_W // _B

_mesh = plsc.VectorSubcoreMesh(core_axis_name="c", subcore_axis_name="s")


@functools.partial(
    pl.kernel,
    out_type=jax.ShapeDtypeStruct((N_EDGES, D_PAD), jnp.float32),
    mesh=_mesh,
    scratch_types=[
        pltpu.VMEM((_B,), jnp.int32),
        pltpu.VMEM((_B,), jnp.int32),
        pltpu.VMEM((_B, D_FEAT), jnp.float32),
        pltpu.VMEM((_B, D_FEAT), jnp.float32),
        pltpu.VMEM((_B, D_EDGE), jnp.float32),
        pltpu.VMEM((_B, D_PAD), jnp.float32),
        pltpu.VMEM((D_GLOBAL,), jnp.float32),
        pltpu.SemaphoreType.DMA,
    ],
)
def _edge_block(edges_hbm, nodes_hbm, global_hbm, recv_hbm, send_hbm, out_hbm,
                ridx, sidx, rbuf, sbuf, ebuf, obuf, gtmp, sem):
    wid = lax.axis_index("s") * _NC + lax.axis_index("c")

    pltpu.sync_copy(global_hbm, gtmp)
    gvec = gtmp[...]
    zvec = jnp.zeros_like(gvec)

    @pl.loop(0, _B)
    def _fill(i):
        obuf[i, pl.ds(D_EDGE + 2 * D_FEAT, D_GLOBAL)] = gvec
        for k in range((D_PAD - D_OUT) // 16):
            obuf[i, pl.ds(D_OUT + 16 * k, 16)] = zvec

    @pl.loop(0, _STEPS)
    def _chunk(step):
        base = wid * _E_PER_W + step * _B
        rows = pl.ds(base, _B)
        pltpu.sync_copy(recv_hbm.at[rows], ridx)
        pltpu.sync_copy(send_hbm.at[rows], sidx)
        pltpu.async_copy(nodes_hbm.at[ridx], rbuf, sem).wait()
        pltpu.async_copy(nodes_hbm.at[sidx], sbuf, sem).wait()
        pltpu.sync_copy(edges_hbm.at[rows], ebuf)

        @pl.loop(0, _B)
        def _assemble(i):
            obuf[i, pl.ds(0, D_EDGE)] = ebuf[i, :]
            for k in range(D_FEAT // 16):
                obuf[i, pl.ds(D_EDGE + 16 * k, 16)] = rbuf[i, pl.ds(16 * k, 16)]
                obuf[i, pl.ds(D_EDGE + D_FEAT + 16 * k, 16)] = sbuf[i, pl.ds(16 * k, 16)]

        pltpu.sync_copy(obuf, out_hbm.at[rows, :])


def kernel(edges_data, nodes_data, global_data, receivers, senders):
    padded = _edge_block(
        edges_data,
        nodes_data,
        global_data,
        receivers.astype(jnp.int32),
        senders.astype(jnp.int32),
    )
    return padded[:, :D_OUT]


# speedup vs baseline: 1.7453x; 1.7453x over previous
import functools

import jax
import jax.numpy as jnp
from jax import lax
from jax.experimental import pallas as pl
from jax.experimental.pallas import tpu as pltpu
from jax.experimental.pallas import tpu_sc as plsc

N_NODES = 10000
N_EDGES = 320000
D_FEAT = 128
D_EDGE = 16
D_GLOBAL = 16
D_OUT = D_EDGE + 2 * D_FEAT + D_GLOBAL
D_PAD = 384

_NC = 2
_NS = 16
_NW = _NC * _NS
_E_PER_W = N_EDGES // _NW
_B = 200
_STEPS = _E_PER_W // _B

_mesh = plsc.VectorSubcoreMesh(core_axis_name="c", subcore_axis_name="s")


@functools.partial(
    pl.kernel,
    out_type=jax.ShapeDtypeStruct((N_EDGES, D_PAD), jnp.float32),
    mesh=_mesh,
    scratch_types=[
        pltpu.VMEM((_B,), jnp.int32),
        pltpu.VMEM((_B,), jnp.int32),
        pltpu.VMEM((_B, D_PAD), jnp.float32),
        pltpu.VMEM((_B, D_EDGE), jnp.float32),
        pltpu.VMEM((D_GLOBAL,), jnp.float32),
        pltpu.SemaphoreType.DMA,
    ],
)
def _edge_block(edges_hbm, rot_hbm, global_hbm, recv_hbm, send_hbm, out_hbm,
                ridx, sidx, obuf, ebuf, gtmp, sem):
    wid = lax.axis_index("s") * _NC + lax.axis_index("c")

    pltpu.sync_copy(global_hbm, gtmp)
    gvec = gtmp[...]
    zvec = jnp.zeros_like(gvec)

    @pl.loop(0, _B)
    def _fill(i):
        obuf[i, pl.ds(D_EDGE + 2 * D_FEAT, D_GLOBAL)] = gvec
        for k in range((D_PAD - D_OUT) // 16):
            obuf[i, pl.ds(D_OUT + 16 * k, 16)] = zvec

    @pl.loop(0, _STEPS)
    def _chunk(step):
        base = wid * _E_PER_W + step * _B
        rows = pl.ds(base, _B)
        pltpu.sync_copy(recv_hbm.at[rows], ridx)
        pltpu.sync_copy(send_hbm.at[rows], sidx)
        pltpu.async_copy(
            rot_hbm.at[ridx], obuf.at[:, pl.ds(0, D_FEAT)], sem).wait()
        pltpu.async_copy(
            rot_hbm.at[sidx], obuf.at[:, pl.ds(D_FEAT, D_FEAT)], sem).wait()
        pltpu.sync_copy(edges_hbm.at[rows], ebuf)

        @pl.loop(0, _B)
        def _assemble(i):
            obuf[i, pl.ds(2 * D_FEAT, 16)] = obuf[i, pl.ds(D_FEAT, 16)]
            obuf[i, pl.ds(D_FEAT, 16)] = obuf[i, pl.ds(0, 16)]
            obuf[i, pl.ds(0, D_EDGE)] = ebuf[i, :]

        pltpu.sync_copy(obuf, out_hbm.at[rows, :])


def kernel(edges_data, nodes_data, global_data, receivers, senders):
    rot = jnp.concatenate(
        [nodes_data[:, D_FEAT - D_EDGE:], nodes_data[:, : D_FEAT - D_EDGE]],
        axis=1)
    padded = _edge_block(
        edges_data,
        rot,
        global_data,
        receivers.astype(jnp.int32),
        senders.astype(jnp.int32),
    )
    return padded[:, :D_OUT]
